# baseline (speedup 1.0000x reference)
import functools

import jax
import jax.numpy as jnp
from jax import lax
from jax.experimental import pallas as pl
from jax.experimental.pallas import tpu as pltpu
from jax.experimental.pallas import tpu_sc as plsc

_NC = 2
_NS = 16
_NW = _NC * _NS
_CHR = 512
_NB = 3


def _make_sc_copy(rows, d):
    rows_per_w = rows // _NW
    nj_per_tensor = rows_per_w // _CHR
    mesh = plsc.VectorSubcoreMesh(
        core_axis_name="c", subcore_axis_name="s",
        num_cores=_NC, num_subcores=_NS)

    @functools.partial(
        pl.kernel,
        out_type=[jax.ShapeDtypeStruct((rows, d), jnp.float16)] * 2,
        mesh=mesh,
        scratch_types=(
            [pltpu.VMEM((_CHR, d), jnp.float16)] * _NB
            + [pltpu.SemaphoreType.DMA] * (2 * _NB)
        ),
    )
    def sc_copy(k_hbm, v_hbm, ko_hbm, vo_hbm,
                b0, b1, b2, si0, si1, si2, so0, so1, so2):
        bufs = (b0, b1, b2)
        sin = (si0, si1, si2)
        sout = (so0, so1, so2)
        wid = lax.axis_index("s") * _NC + lax.axis_index("c")
        base = wid * rows_per_w

        jobs = []
        for src, dst in ((k_hbm, ko_hbm), (v_hbm, vo_hbm)):
            for c in range(nj_per_tensor):
                jobs.append((src, dst, c * _CHR))
        ins, outs = [], []
        for j, (src, dst, off) in enumerate(jobs):
            b = j % _NB
            sl = pl.ds(base + off, _CHR)
            ins.append(pltpu.make_async_copy(src.at[sl], bufs[b], sin[b]))
            outs.append(pltpu.make_async_copy(bufs[b], dst.at[sl], sout[b]))

        nj = len(jobs)
        for j in range(min(_NB - 1, nj)):
            ins[j].start()
        waited = set()
        for j in range(nj):
            ins[j].wait()
            outs[j].start()
            nxt = j + _NB - 1
            if nxt < nj:
                if j >= 1:
                    outs[j - 1].wait()
                    waited.add(j - 1)
                ins[nxt].start()
        for j in range(nj):
            if j not in waited:
                outs[j].wait()

    return sc_copy


def kernel(key_states, value_states, k_cache, v_cache, layer_idx):
    B, H, S, D = key_states.shape
    rows = B * H * S
    k = key_states.reshape(rows, D)
    v = value_states.reshape(rows, D)
    ko, vo = _make_sc_copy(rows, D)(k, v)
    return ko.reshape(B, H, S, D), vo.reshape(B, H, S, D)


# speedup vs baseline: 1.0370x; 1.0370x over previous
import functools

import jax
import jax.numpy as jnp
from jax import lax
from jax.experimental import pallas as pl
from jax.experimental.pallas import tpu as pltpu
from jax.experimental.pallas import tpu_sc as plsc

_NC = 2
_NS = 16
_NW = _NC * _NS
_CHR = 512
_NB = 3


def _make_sc_copy(rows, d):
    rows_per_w = rows // _NW
    nj_per_tensor = rows_per_w // _CHR
    mesh = plsc.VectorSubcoreMesh(
        core_axis_name="c", subcore_axis_name="s",
        num_cores=_NC, num_subcores=_NS)

    @functools.partial(
        pl.kernel,
        out_type=[jax.ShapeDtypeStruct((rows, d), jnp.float16)] * 2,
        mesh=mesh,
        scratch_types=(
            [pltpu.VMEM((_CHR, d), jnp.float16)] * _NB
            + [pltpu.SemaphoreType.DMA] * (2 * _NB)
        ),
    )
    def sc_copy(k_hbm, v_hbm, ko_hbm, vo_hbm,
                b0, b1, b2, si0, si1, si2, so0, so1, so2):
        bufs = (b0, b1, b2)
        sin = (si0, si1, si2)
        sout = (so0, so1, so2)
        wid = lax.axis_index("s") * _NC + lax.axis_index("c")
        base = wid * rows_per_w

        jobs = []
        for src, dst in ((k_hbm, ko_hbm), (v_hbm, vo_hbm)):
            for c in range(nj_per_tensor):
                jobs.append((src, dst, c * _CHR))
        ins, outs = [], []
        for j, (src, dst, off) in enumerate(jobs):
            b = j % _NB
            sl = pl.ds(base + off, _CHR)
            ins.append(pltpu.make_async_copy(src.at[sl], bufs[b], sin[b]))
            outs.append(pltpu.make_async_copy(bufs[b], dst.at[sl], sout[b]))

        nj = len(jobs)
        for j in range(min(_NB, nj)):
            ins[j].start()
        for j in range(nj):
            ins[j].wait()
            outs[j].start()
            nxt = j + _NB
            if nxt < nj:
                outs[j].wait()
                ins[nxt].start()
        for j in range(max(0, nj - _NB), nj):
            outs[j].wait()

    return sc_copy


def kernel(key_states, value_states, k_cache, v_cache, layer_idx):
    B, H, S, D = key_states.shape
    rows = B * H * S
    k = key_states.reshape(rows, D)
    v = value_states.reshape(rows, D)
    ko, vo = _make_sc_copy(rows, D)(k, v)
    return ko.reshape(B, H, S, D), vo.reshape(B, H, S, D)
